# Initial kernel scaffold; baseline (speedup 1.0000x reference)
import jax
import jax.numpy as jnp
from jax.experimental import pallas as pl


def kernel(in_states, edges, W_msg, att_w, passthrough_coef):
    raise NotImplementedError("write your pallas kernel here")


# speedup vs baseline: 17.3517x; 17.3517x over previous
import jax
import jax.numpy as jnp
from jax import lax
from jax.experimental import pallas as pl
from jax.experimental.pallas import tpu as pltpu
from jax.experimental.pallas import tpu_sc as plsc

N_NODES = 10000
N_EDGES = 320000
D = 128

NC = 2
NS = 16
NW = NC * NS
L = 16

NPAD = 10240
E2 = N_EDGES // NW
C = 80
NROWS = NPAD // NS


def _sc_body(src_hbm, dst_hbm, sa_hbm, sb_hbm, msg_hbm,
             part_hbm, lsums_hbm,
             sa_v, sb_v, lsum_v, srcc_v, dstc_v, coef_v, rows_v,
             acc_sh, sem):
  cid = lax.axis_index("c")
  sid = lax.axis_index("s")
  wid = cid * NS + sid

  zeros16 = jnp.zeros((L,), jnp.float32)

  pltpu.sync_copy(sa_hbm, sa_v)
  pltpu.sync_copy(sb_hbm, sb_v)

  def zero_lsum(i, carry):
    lsum_v[pl.ds(i * L, L)] = zeros16
    return carry
  lax.fori_loop(0, NPAD // L, zero_lsum, 0)

  def zero_rows(i, carry):
    rows_v[i // (D // L), pl.ds((i % (D // L)) * L, L)] = zeros16
    return carry
  lax.fori_loop(0, C * D // L, zero_rows, 0)

  ab = sid * NROWS
  for k in range(NROWS // C):
    pltpu.sync_copy(rows_v, acc_sh.at[pl.ds(ab + k * C, C)])
  plsc.subcore_barrier()

  base2 = wid * E2

  def ch_body(k, carry):
    eb = base2 + k * C
    pltpu.sync_copy(src_hbm.at[pl.ds(eb, C)], srcc_v)
    pltpu.sync_copy(dst_hbm.at[pl.ds(eb, C)], dstc_v)
    cp = pltpu.async_copy(msg_hbm.at[srcc_v], rows_v, sem)
    def coef_body(g, carry2):
      s16 = srcc_v[pl.ds(g * L, L)]
      d16 = dstc_v[pl.ds(g * L, L)]
      a = plsc.load_gather(sa_v, [s16])
      b = plsc.load_gather(sb_v, [d16])
      x = a + b
      lr = jnp.where(x >= 0.0, x, 0.01 * x)
      p = jnp.exp(lr)
      plsc.addupdate_scatter(lsum_v, [d16], p)
      coef_v[pl.ds(g * L, L)] = p
      return carry2
    lax.fori_loop(0, C // L, coef_body, 0)
    cp.wait()

    def scale_group(g, carry2):
      cvec = coef_v[pl.ds(g * L, L)]
      for i in range(L):
        c = cvec[i]
        e = g * L + i
        for j in range(D // L):
          rows_v[e, pl.ds(j * L, L)] = rows_v[e, pl.ds(j * L, L)] * c
      return carry2
    lax.fori_loop(0, C // L, scale_group, 0)

    pltpu.sync_copy(rows_v, acc_sh.at[dstc_v], add=True)
    return carry
  lax.fori_loop(0, E2 // C, ch_body, 0)

  pltpu.sync_copy(lsum_v, lsums_hbm.at[wid])
  plsc.subcore_barrier()
  pltpu.sync_copy(acc_sh.at[pl.ds(ab, NROWS)],
                  part_hbm.at[cid, pl.ds(ab, NROWS)])


_sc_aggregate = pl.kernel(
    _sc_body,
    out_type=[
        jax.ShapeDtypeStruct((NC, NPAD, D), jnp.float32),
        jax.ShapeDtypeStruct((NW, NPAD), jnp.float32),
    ],
    mesh=plsc.VectorSubcoreMesh(core_axis_name="c", subcore_axis_name="s",
                                num_cores=NC, num_subcores=NS),
    compiler_params=pltpu.CompilerParams(needs_layout_passes=False),
    scratch_types=[
        pltpu.VMEM((N_NODES,), jnp.float32),
        pltpu.VMEM((N_NODES,), jnp.float32),
        pltpu.VMEM((NPAD,), jnp.float32),
        pltpu.VMEM((C,), jnp.int32),
        pltpu.VMEM((C,), jnp.int32),
        pltpu.VMEM((C,), jnp.float32),
        pltpu.VMEM((C, D), jnp.float32),
        pltpu.VMEM_SHARED((NPAD, D), jnp.float32),
        pltpu.SemaphoreType.DMA,
    ],
)


def _tc1_body(x_ref, w_ref, a_ref, m_ref, s_ref):
  x = x_ref[...]
  m = lax.dot_general(x, w_ref[...], (((1,), (1,)), ((), ())),
                      preferred_element_type=jnp.float32)
  m_ref[...] = m
  s_ref[...] = lax.dot_general(m, a_ref[...], (((1,), (0,)), ((), ())),
                               preferred_element_type=jnp.float32)


_tc_messages = pl.pallas_call(
    _tc1_body,
    out_shape=[
        jax.ShapeDtypeStruct((N_NODES, D), jnp.float32),
        jax.ShapeDtypeStruct((N_NODES, 8), jnp.float32),
    ],
)


def _tc2_body(part_ref, ls_ref, x_ref, c_ref, o_ref):
  sig = jax.nn.sigmoid(c_ref[0])
  sums = jnp.sum(ls_ref[...], axis=0)[0:N_NODES]
  sums = jnp.where(sums == 0.0, 1.0, sums)
  merged = (part_ref[0, 0:N_NODES, :] + part_ref[1, 0:N_NODES, :]) / sums[:, None]
  o_ref[...] = jnp.maximum(merged, 0.0) + x_ref[...] * sig


_tc_finish = pl.pallas_call(
    _tc2_body,
    in_specs=[
        pl.BlockSpec(),
        pl.BlockSpec(),
        pl.BlockSpec(),
        pl.BlockSpec(memory_space=pltpu.SMEM),
    ],
    out_shape=jax.ShapeDtypeStruct((N_NODES, D), jnp.float32),
)


@jax.jit
def kernel(in_states, edges, W_msg, att_w, passthrough_coef):
  src = edges[0]
  dst = edges[1]
  a_pad = jnp.zeros((D, 8), jnp.float32)
  a_pad = a_pad.at[:, 0].set(att_w[0, :D]).at[:, 1].set(att_w[0, D:])
  messages, s = _tc_messages(in_states, W_msg, a_pad)
  sa = s[:, 0]
  sb = s[:, 1]
  parts, lsums = _sc_aggregate(src, dst, sa, sb, messages)
  out = _tc_finish(parts, lsums, in_states, passthrough_coef.reshape(1))
  return out
